# baseline (speedup 1.0000x reference)
import functools

import jax
import jax.numpy as jnp
from jax import lax
from jax.experimental import pallas as pl
from jax.experimental.pallas import tpu as pltpu
from jax.experimental.pallas import tpu_sc as plsc

N = 10000
D = 256
DH = 128
E = 160000
CHUNK = 128
NSUB = 16
NCORE = 2

AGG_CH = 80
BLK = 16
NBLK = AGG_CH // BLK
EPAD = NSUB * AGG_CH * CHUNK
NACC = N + 8
RPT = 624
RPT_LAST = 640


def _mesh():
    return plsc.VectorSubcoreMesh(core_axis_name="c", subcore_axis_name="s")


def _sc_aggregate(g2, src2, dst2):
    @functools.partial(
        pl.kernel,
        mesh=_mesh(),
        out_type=jax.ShapeDtypeStruct((NCORE * N, DH), jnp.float32),
        scratch_types=[
            pltpu.VMEM((BLK, CHUNK), jnp.int32),
            pltpu.VMEM((BLK, CHUNK), jnp.int32),
            pltpu.VMEM((CHUNK, DH), jnp.float32),
            pltpu.VMEM((CHUNK, DH), jnp.float32),
            pltpu.SemaphoreType.DMA,
            pltpu.SemaphoreType.DMA,
            pltpu.VMEM_SHARED((NACC, DH), jnp.float32),
        ],
    )
    def k(g_hbm, src_hbm, dst_hbm, out_hbm, sidx, didx, buf_a, buf_b,
          sem_a, sem_b, acc):
        c = lax.axis_index("c")
        s = lax.axis_index("s")

        @pl.when(s < NSUB - 1)
        def _():
            pltpu.sync_copy(g_hbm.at[pl.ds(c * N + s * RPT, RPT)],
                            acc.at[pl.ds(s * RPT, RPT)])

        @pl.when(s == NSUB - 1)
        def _():
            pltpu.sync_copy(g_hbm.at[pl.ds(c * N + (NSUB - 1) * RPT, RPT_LAST)],
                            acc.at[pl.ds((NSUB - 1) * RPT, RPT_LAST)])

        plsc.subcore_barrier()
        sbase = (c * NSUB + s) * AGG_CH
        dbase = s * AGG_CH

        H = CHUNK // 2

        def gather2(k, buf, sem):
            pltpu.async_copy(g_hbm.at[sidx.at[k, pl.ds(0, H)]],
                             buf.at[pl.ds(0, H)], sem)
            pltpu.async_copy(g_hbm.at[sidx.at[k, pl.ds(H, H)]],
                             buf.at[pl.ds(H, H)], sem)

        def block(b, carry):
            pltpu.sync_copy(src_hbm.at[pl.ds(sbase + b * BLK, BLK)], sidx)
            pltpu.sync_copy(dst_hbm.at[pl.ds(dbase + b * BLK, BLK)], didx)
            gather2(0, buf_a, sem_a)

            def pair(j, carry2):
                gather2(2 * j + 1, buf_b, sem_b)
                pltpu.make_async_copy(g_hbm.at[pl.ds(0, CHUNK)], buf_a,
                                      sem_a).wait()
                pltpu.sync_copy(buf_a, acc.at[didx.at[2 * j]], add=True)
                gather2(2 * j + 2, buf_a, sem_a)
                pltpu.make_async_copy(g_hbm.at[pl.ds(0, CHUNK)], buf_b,
                                      sem_b).wait()
                pltpu.sync_copy(buf_b, acc.at[didx.at[2 * j + 1]], add=True)
                return carry2

            lax.fori_loop(0, BLK // 2 - 1, pair, 0)
            gather2(BLK - 1, buf_b, sem_b)
            pltpu.make_async_copy(g_hbm.at[pl.ds(0, CHUNK)], buf_a, sem_a).wait()
            pltpu.sync_copy(buf_a, acc.at[didx.at[BLK - 2]], add=True)
            pltpu.make_async_copy(g_hbm.at[pl.ds(0, CHUNK)], buf_b, sem_b).wait()
            pltpu.sync_copy(buf_b, acc.at[didx.at[BLK - 1]], add=True)
            return carry

        lax.fori_loop(0, NBLK, block, 0)
        plsc.subcore_barrier()

        @pl.when(s < NSUB - 1)
        def _():
            pltpu.sync_copy(acc.at[pl.ds(s * RPT, RPT)],
                            out_hbm.at[pl.ds(c * N + s * RPT, RPT)])

        @pl.when(s == NSUB - 1)
        def _():
            pltpu.sync_copy(acc.at[pl.ds((NSUB - 1) * RPT, RPT_LAST)],
                            out_hbm.at[pl.ds(c * N + (NSUB - 1) * RPT, RPT_LAST)])

    return k(g2, src2, dst2)


DEG_CH = (EPAD // CHUNK) // (NSUB * NCORE)


def _sc_degree(ones_g, dst2, ones_row):
    @functools.partial(
        pl.kernel,
        mesh=_mesh(),
        out_type=jax.ShapeDtypeStruct((NCORE * N, DH), jnp.float32),
        scratch_types=[
            pltpu.VMEM((DEG_CH, CHUNK), jnp.int32),
            pltpu.VMEM((CHUNK, DH), jnp.float32),
            pltpu.SemaphoreType.DMA,
            pltpu.VMEM_SHARED((NACC, DH), jnp.float32),
        ],
    )
    def k(g_hbm, dst_hbm, ones_hbm, out_hbm, didx, ones_v, sem, acc):
        c = lax.axis_index("c")
        s = lax.axis_index("s")
        wid = s * NCORE + c
        pltpu.sync_copy(dst_hbm.at[pl.ds(wid * DEG_CH, DEG_CH)], didx)
        pltpu.sync_copy(ones_hbm, ones_v)

        @pl.when(s < NSUB - 1)
        def _():
            pltpu.sync_copy(g_hbm.at[pl.ds(s * RPT, RPT)],
                            acc.at[pl.ds(s * RPT, RPT)])

        @pl.when(s == NSUB - 1)
        def _():
            pltpu.sync_copy(g_hbm.at[pl.ds((NSUB - 1) * RPT, RPT_LAST)],
                            acc.at[pl.ds((NSUB - 1) * RPT, RPT_LAST)])

        plsc.subcore_barrier()

        def chunk(j, carry):
            pltpu.async_copy(ones_v, acc.at[didx.at[j]], sem, add=True)
            return carry

        lax.fori_loop(0, DEG_CH, chunk, 0)

        def dr(j, carry):
            pltpu.make_async_copy(ones_hbm, ones_v, sem).wait()
            return carry

        lax.fori_loop(0, DEG_CH, dr, 0)
        plsc.subcore_barrier()

        @pl.when(s < NSUB - 1)
        def _():
            pltpu.sync_copy(acc.at[pl.ds(s * RPT, RPT)],
                            out_hbm.at[pl.ds(c * N + s * RPT, RPT)])

        @pl.when(s == NSUB - 1)
        def _():
            pltpu.sync_copy(acc.at[pl.ds((NSUB - 1) * RPT, RPT_LAST)],
                            out_hbm.at[pl.ds(c * N + (NSUB - 1) * RPT, RPT_LAST)])

    return k(ones_g, dst2, ones_row)


def _mm1_body(x_ref, w_ref, o_ref):
    o_ref[0] = jnp.dot(x_ref[...], w_ref[0],
                       preferred_element_type=jnp.float32)


def _tc_matmul(x, Wh):
    Rb = 1000
    return pl.pallas_call(
        _mm1_body,
        grid=(NCORE, N // Rb),
        in_specs=[
            pl.BlockSpec((Rb, D), lambda c, r: (r, 0)),
            pl.BlockSpec((1, D, DH), lambda c, r: (c, 0, 0)),
        ],
        out_specs=pl.BlockSpec((1, Rb, DH), lambda c, r: (c, r, 0)),
        out_shape=jax.ShapeDtypeStruct((NCORE, N, DH), jnp.float32),
    )(x, Wh)


def _dinv_scale_body(deg_ref, h_ref, dv_ref, g_ref):
    dv = lax.rsqrt(deg_ref[0] + deg_ref[1] - 1.0)
    dv_ref[...] = dv
    for c in range(NCORE):
        g_ref[c] = h_ref[c] * dv


def _tc_dinv_scale(deg, h1):
    Rb = 1000
    return pl.pallas_call(
        _dinv_scale_body,
        grid=(N // Rb,),
        in_specs=[
            pl.BlockSpec((NCORE, Rb, DH), lambda i: (0, i, 0)),
            pl.BlockSpec((NCORE, Rb, DH), lambda i: (0, i, 0)),
        ],
        out_specs=[
            pl.BlockSpec((Rb, DH), lambda i: (i, 0)),
            pl.BlockSpec((NCORE, Rb, DH), lambda i: (0, i, 0)),
        ],
        out_shape=[
            jax.ShapeDtypeStruct((N, DH), jnp.float32),
            jax.ShapeDtypeStruct((NCORE, N, DH), jnp.float32),
        ],
    )(deg, h1)


def _mm2_body(a_ref, d_ref, b_ref, w_ref, o_ref):
    d = d_ref[...]
    x2 = jnp.concatenate(
        [jnp.maximum(a_ref[h] * d + b_ref[h:h + 1, :], 0.0)
         for h in range(NCORE)], axis=1)
    o_ref[0] = jnp.dot(x2, w_ref[0], preferred_element_type=jnp.float32) * d


def _tc_mm2_fused(agg1, dinv2d, b2d, Wh):
    Rb = 1000
    return pl.pallas_call(
        _mm2_body,
        grid=(NCORE, N // Rb),
        in_specs=[
            pl.BlockSpec((NCORE, Rb, DH), lambda c, r: (0, r, 0)),
            pl.BlockSpec((Rb, DH), lambda c, r: (r, 0)),
            pl.BlockSpec((NCORE, DH), lambda c, r: (0, 0)),
            pl.BlockSpec((1, D, DH), lambda c, r: (c, 0, 0)),
        ],
        out_specs=pl.BlockSpec((1, Rb, DH), lambda c, r: (c, r, 0)),
        out_shape=jax.ShapeDtypeStruct((NCORE, N, DH), jnp.float32),
    )(agg1, dinv2d, b2d, Wh)


def _epi2_body(a_ref, d_ref, b_ref, o_ref):
    d = d_ref[...]
    hs = [a_ref[h] * d + b_ref[h:h + 1, :] for h in range(NCORE)]
    m = jnp.maximum(hs[0].max(axis=1, keepdims=True),
                    hs[1].max(axis=1, keepdims=True))
    ssum = (jnp.exp(hs[0] - m).sum(axis=1, keepdims=True)
            + jnp.exp(hs[1] - m).sum(axis=1, keepdims=True))
    lse = m + jnp.log(ssum)
    for h in range(NCORE):
        o_ref[:, h * DH:(h + 1) * DH] = hs[h] - lse


def _tc_epilogue(agg, dinv2d, b2d, body):
    Rb = 1000
    return pl.pallas_call(
        body,
        grid=(N // Rb,),
        in_specs=[
            pl.BlockSpec((NCORE, Rb, DH), lambda r: (0, r, 0)),
            pl.BlockSpec((Rb, DH), lambda r: (r, 0)),
            pl.BlockSpec((NCORE, DH), lambda r: (0, 0)),
        ],
        out_specs=pl.BlockSpec((Rb, D), lambda r: (r, 0)),
        out_shape=jax.ShapeDtypeStruct((N, D), jnp.float32),
    )(agg, dinv2d, b2d)


def kernel(x, edge_index, W1, b1, W2, b2):
    src = edge_index[0].astype(jnp.int32)
    dst = edge_index[1].astype(jnp.int32)
    pad = EPAD - E
    srcp = jnp.concatenate([src, jnp.zeros((pad,), jnp.int32)])
    dstp = jnp.concatenate([dst, jnp.full((pad,), N, jnp.int32)])
    src2 = jnp.concatenate([srcp, srcp + N]).reshape(NCORE * EPAD // CHUNK,
                                                     CHUNK)
    dst2 = dstp.reshape(EPAD // CHUNK, CHUNK)

    ones_g = jnp.ones((N, DH), jnp.float32)
    ones_row = jnp.ones((CHUNK, DH), jnp.float32)
    W1h = W1.reshape(D, NCORE, DH).transpose(1, 0, 2)
    W2h = W2.reshape(D, NCORE, DH).transpose(1, 0, 2)

    deg = _sc_degree(ones_g, dst2, ones_row).reshape(NCORE, N, DH)
    h1 = _tc_matmul(x, W1h)
    dinv2d, g1 = _tc_dinv_scale(deg, h1)

    agg1 = _sc_aggregate(g1.reshape(NCORE * N, DH), src2, dst2)
    g2 = _tc_mm2_fused(agg1.reshape(NCORE, N, DH), dinv2d,
                       b1.reshape(NCORE, DH), W2h)

    agg2 = _sc_aggregate(g2.reshape(NCORE * N, DH), src2, dst2)
    return _tc_epilogue(agg2.reshape(NCORE, N, DH), dinv2d,
                        b2.reshape(NCORE, DH), _epi2_body)


# speedup vs baseline: 1.0210x; 1.0210x over previous
import functools

import jax
import jax.numpy as jnp
from jax import lax
from jax.experimental import pallas as pl
from jax.experimental.pallas import tpu as pltpu
from jax.experimental.pallas import tpu_sc as plsc

N = 10000
D = 256
DH = 128
E = 160000
CHUNK = 128
NSUB = 16
NCORE = 2

AGG_CH = 80
BLK = 16
NBLK = AGG_CH // BLK
EPAD = NSUB * AGG_CH * CHUNK
NACC = N + 8
RPT = 624
RPT_LAST = 640


def _mesh():
    return plsc.VectorSubcoreMesh(core_axis_name="c", subcore_axis_name="s")


def _sc_aggregate(g2, src2, dst2):
    @functools.partial(
        pl.kernel,
        mesh=_mesh(),
        out_type=jax.ShapeDtypeStruct((NCORE * N, DH), jnp.float32),
        scratch_types=[
            pltpu.VMEM((BLK, CHUNK), jnp.int32),
            pltpu.VMEM((BLK, CHUNK), jnp.int32),
            pltpu.VMEM((CHUNK, DH), jnp.float32),
            pltpu.VMEM((CHUNK, DH), jnp.float32),
            pltpu.SemaphoreType.DMA,
            pltpu.SemaphoreType.DMA,
            pltpu.VMEM_SHARED((NACC, DH), jnp.float32),
        ],
    )
    def k(g_hbm, src_hbm, dst_hbm, out_hbm, sidx, didx, buf_a, buf_b,
          sem_a, sem_b, acc):
        c = lax.axis_index("c")
        s = lax.axis_index("s")

        @pl.when(s < NSUB - 1)
        def _():
            pltpu.sync_copy(g_hbm.at[pl.ds(c * N + s * RPT, RPT)],
                            acc.at[pl.ds(s * RPT, RPT)])

        @pl.when(s == NSUB - 1)
        def _():
            pltpu.sync_copy(g_hbm.at[pl.ds(c * N + (NSUB - 1) * RPT, RPT_LAST)],
                            acc.at[pl.ds((NSUB - 1) * RPT, RPT_LAST)])

        plsc.subcore_barrier()
        sbase = (c * NSUB + s) * AGG_CH
        dbase = s * AGG_CH

        def block(b, carry):
            pltpu.sync_copy(src_hbm.at[pl.ds(sbase + b * BLK, BLK)], sidx)
            pltpu.sync_copy(dst_hbm.at[pl.ds(dbase + b * BLK, BLK)], didx)
            pltpu.async_copy(g_hbm.at[sidx.at[0]], buf_a, sem_a)

            def pair(j, carry2):
                pltpu.async_copy(g_hbm.at[sidx.at[2 * j + 1]], buf_b, sem_b)
                pltpu.make_async_copy(g_hbm.at[pl.ds(0, CHUNK)], buf_a,
                                      sem_a).wait()
                pltpu.sync_copy(buf_a, acc.at[didx.at[2 * j]], add=True)
                pltpu.async_copy(g_hbm.at[sidx.at[2 * j + 2]], buf_a, sem_a)
                pltpu.make_async_copy(g_hbm.at[pl.ds(0, CHUNK)], buf_b,
                                      sem_b).wait()
                pltpu.sync_copy(buf_b, acc.at[didx.at[2 * j + 1]], add=True)
                return carry2

            lax.fori_loop(0, BLK // 2 - 1, pair, 0)
            pltpu.async_copy(g_hbm.at[sidx.at[BLK - 1]], buf_b, sem_b)
            pltpu.make_async_copy(g_hbm.at[pl.ds(0, CHUNK)], buf_a, sem_a).wait()
            pltpu.sync_copy(buf_a, acc.at[didx.at[BLK - 2]], add=True)
            pltpu.make_async_copy(g_hbm.at[pl.ds(0, CHUNK)], buf_b, sem_b).wait()
            pltpu.sync_copy(buf_b, acc.at[didx.at[BLK - 1]], add=True)
            return carry

        lax.fori_loop(0, NBLK, block, 0)
        plsc.subcore_barrier()

        @pl.when(s < NSUB - 1)
        def _():
            pltpu.sync_copy(acc.at[pl.ds(s * RPT, RPT)],
                            out_hbm.at[pl.ds(c * N + s * RPT, RPT)])

        @pl.when(s == NSUB - 1)
        def _():
            pltpu.sync_copy(acc.at[pl.ds((NSUB - 1) * RPT, RPT_LAST)],
                            out_hbm.at[pl.ds(c * N + (NSUB - 1) * RPT, RPT_LAST)])

    return k(g2, src2, dst2)


DEG_CH = (EPAD // CHUNK) // (NSUB * NCORE)


def _sc_degree(ones_g, dst2, ones_row):
    @functools.partial(
        pl.kernel,
        mesh=_mesh(),
        out_type=jax.ShapeDtypeStruct((NCORE * N, DH), jnp.float32),
        scratch_types=[
            pltpu.VMEM((DEG_CH, CHUNK), jnp.int32),
            pltpu.VMEM((CHUNK, DH), jnp.float32),
            pltpu.SemaphoreType.DMA,
            pltpu.VMEM_SHARED((NACC, DH), jnp.float32),
        ],
    )
    def k(g_hbm, dst_hbm, ones_hbm, out_hbm, didx, ones_v, sem, acc):
        c = lax.axis_index("c")
        s = lax.axis_index("s")
        wid = s * NCORE + c
        pltpu.sync_copy(dst_hbm.at[pl.ds(wid * DEG_CH, DEG_CH)], didx)
        pltpu.sync_copy(ones_hbm, ones_v)

        @pl.when(s < NSUB - 1)
        def _():
            pltpu.sync_copy(g_hbm.at[pl.ds(s * RPT, RPT)],
                            acc.at[pl.ds(s * RPT, RPT)])

        @pl.when(s == NSUB - 1)
        def _():
            pltpu.sync_copy(g_hbm.at[pl.ds((NSUB - 1) * RPT, RPT_LAST)],
                            acc.at[pl.ds((NSUB - 1) * RPT, RPT_LAST)])

        plsc.subcore_barrier()

        def chunk(j, carry):
            pltpu.async_copy(ones_v, acc.at[didx.at[j]], sem, add=True)
            return carry

        lax.fori_loop(0, DEG_CH, chunk, 0)

        def dr(j, carry):
            pltpu.make_async_copy(ones_hbm, ones_v, sem).wait()
            return carry

        lax.fori_loop(0, DEG_CH, dr, 0)
        plsc.subcore_barrier()

        @pl.when(s < NSUB - 1)
        def _():
            pltpu.sync_copy(acc.at[pl.ds(s * RPT, RPT)],
                            out_hbm.at[pl.ds(c * N + s * RPT, RPT)])

        @pl.when(s == NSUB - 1)
        def _():
            pltpu.sync_copy(acc.at[pl.ds((NSUB - 1) * RPT, RPT_LAST)],
                            out_hbm.at[pl.ds(c * N + (NSUB - 1) * RPT, RPT_LAST)])

    return k(ones_g, dst2, ones_row)


def _mm1_body(x_ref, w_ref, o_ref):
    o_ref[0] = jnp.dot(x_ref[...], w_ref[0],
                       preferred_element_type=jnp.float32)


def _tc_matmul(x, Wh):
    Rb = 1000
    return pl.pallas_call(
        _mm1_body,
        grid=(NCORE, N // Rb),
        in_specs=[
            pl.BlockSpec((Rb, D), lambda c, r: (r, 0)),
            pl.BlockSpec((1, D, DH), lambda c, r: (c, 0, 0)),
        ],
        out_specs=pl.BlockSpec((1, Rb, DH), lambda c, r: (c, r, 0)),
        out_shape=jax.ShapeDtypeStruct((NCORE, N, DH), jnp.float32),
    )(x, Wh)


def _dinv_scale_body(deg_ref, h_ref, dv_ref, g_ref):
    dv = lax.rsqrt(deg_ref[0] + deg_ref[1] - 1.0)
    dv_ref[...] = dv
    for c in range(NCORE):
        g_ref[c] = h_ref[c] * dv


def _tc_dinv_scale(deg, h1):
    Rb = 1000
    return pl.pallas_call(
        _dinv_scale_body,
        grid=(N // Rb,),
        in_specs=[
            pl.BlockSpec((NCORE, Rb, DH), lambda i: (0, i, 0)),
            pl.BlockSpec((NCORE, Rb, DH), lambda i: (0, i, 0)),
        ],
        out_specs=[
            pl.BlockSpec((Rb, DH), lambda i: (i, 0)),
            pl.BlockSpec((NCORE, Rb, DH), lambda i: (0, i, 0)),
        ],
        out_shape=[
            jax.ShapeDtypeStruct((N, DH), jnp.float32),
            jax.ShapeDtypeStruct((NCORE, N, DH), jnp.float32),
        ],
    )(deg, h1)


def _mm2_body(a_ref, d_ref, b_ref, w_ref, o_ref):
    d = d_ref[...]
    x2 = jnp.concatenate(
        [jnp.maximum(a_ref[h] * d + b_ref[h:h + 1, :], 0.0)
         for h in range(NCORE)], axis=1)
    o_ref[0] = jnp.dot(x2, w_ref[0], preferred_element_type=jnp.float32) * d


def _tc_mm2_fused(agg1, dinv2d, b2d, Wh):
    Rb = 1000
    return pl.pallas_call(
        _mm2_body,
        grid=(NCORE, N // Rb),
        in_specs=[
            pl.BlockSpec((NCORE, Rb, DH), lambda c, r: (0, r, 0)),
            pl.BlockSpec((Rb, DH), lambda c, r: (r, 0)),
            pl.BlockSpec((NCORE, DH), lambda c, r: (0, 0)),
            pl.BlockSpec((1, D, DH), lambda c, r: (c, 0, 0)),
        ],
        out_specs=pl.BlockSpec((1, Rb, DH), lambda c, r: (c, r, 0)),
        out_shape=jax.ShapeDtypeStruct((NCORE, N, DH), jnp.float32),
    )(agg1, dinv2d, b2d, Wh)


def _epi2_body(a_ref, d_ref, b_ref, o_ref):
    d = d_ref[...]
    hs = [a_ref[h] * d + b_ref[h:h + 1, :] for h in range(NCORE)]
    m = jnp.maximum(hs[0].max(axis=1, keepdims=True),
                    hs[1].max(axis=1, keepdims=True))
    ssum = (jnp.exp(hs[0] - m).sum(axis=1, keepdims=True)
            + jnp.exp(hs[1] - m).sum(axis=1, keepdims=True))
    lse = m + jnp.log(ssum)
    for h in range(NCORE):
        o_ref[:, h * DH:(h + 1) * DH] = hs[h] - lse


def _tc_epilogue(agg, dinv2d, b2d, body):
    Rb = 1000
    return pl.pallas_call(
        body,
        grid=(N // Rb,),
        in_specs=[
            pl.BlockSpec((NCORE, Rb, DH), lambda r: (0, r, 0)),
            pl.BlockSpec((Rb, DH), lambda r: (r, 0)),
            pl.BlockSpec((NCORE, DH), lambda r: (0, 0)),
        ],
        out_specs=pl.BlockSpec((Rb, D), lambda r: (r, 0)),
        out_shape=jax.ShapeDtypeStruct((N, D), jnp.float32),
    )(agg, dinv2d, b2d)


def kernel(x, edge_index, W1, b1, W2, b2):
    src = edge_index[0].astype(jnp.int32)
    dst = edge_index[1].astype(jnp.int32)
    pad = EPAD - E
    srcp = jnp.concatenate([src, jnp.zeros((pad,), jnp.int32)])
    dstp = jnp.concatenate([dst, jnp.full((pad,), N, jnp.int32)])
    src2 = jnp.concatenate([srcp, srcp + N]).reshape(NCORE * EPAD // CHUNK,
                                                     CHUNK)
    dst2 = dstp.reshape(EPAD // CHUNK, CHUNK)

    ones_g = jnp.ones((N, DH), jnp.float32)
    ones_row = jnp.ones((CHUNK, DH), jnp.float32)
    W1h = W1.reshape(D, NCORE, DH).transpose(1, 0, 2)
    W2h = W2.reshape(D, NCORE, DH).transpose(1, 0, 2)

    deg = _sc_degree(ones_g, dst2, ones_row).reshape(NCORE, N, DH)
    h1 = _tc_matmul(x, W1h)
    dinv2d, g1 = _tc_dinv_scale(deg, h1)

    agg1 = _sc_aggregate(g1.reshape(NCORE * N, DH), src2, dst2)
    g2 = _tc_mm2_fused(agg1.reshape(NCORE, N, DH), dinv2d,
                       b1.reshape(NCORE, DH), W2h)

    agg2 = _sc_aggregate(g2.reshape(NCORE * N, DH), src2, dst2)
    return _tc_epilogue(agg2.reshape(NCORE, N, DH), dinv2d,
                        b2.reshape(NCORE, DH), _epi2_body)
